# baseline (speedup 1.0000x reference)
import jax
import jax.numpy as jnp
from jax.experimental import pallas as pl

_L1 = 128
_L2 = 64
_L3 = 32
_BM = 2048
_K = 256


def _dense_from_coo(w_ref, rows_ref, cols_ref, in_dim):
    c_iota = jax.lax.broadcasted_iota(jnp.int32, (in_dim, _K), 0)
    cw = jnp.where(cols_ref[0, :][None, :] == c_iota, w_ref[0, :][None, :], 0.0)
    r_iota = jax.lax.broadcasted_iota(jnp.int32, (128, _K), 0)
    r1h = jnp.where(rows_ref[0, :][None, :] == r_iota, 1.0, 0.0)
    return jax.lax.dot_general(
        cw, r1h, (((1,), (1,)), ((), ())),
        preferred_element_type=jnp.float32,
        precision=jax.lax.Precision.HIGHEST,
    )


def _densify(w1_ref, r1_ref, c1_ref, w2_ref, r2_ref, c2_ref, w3_ref, r3_ref,
             c3_ref, w4_ref, r4_ref, c4_ref, m1_ref, m2_ref, m3_ref, m4_ref):
    m1_ref[...] = _dense_from_coo(w1_ref, r1_ref, c1_ref, _L1)
    m2_ref[...] = _dense_from_coo(w2_ref, r2_ref, c2_ref, 128)
    m3_ref[...] = _dense_from_coo(w3_ref, r3_ref, c3_ref, 128)
    m4_ref[...] = _dense_from_coo(w4_ref, r4_ref, c4_ref, 128)


def _hnn_block(x_ref, m1_ref, m2_ref, m3_ref, m4_ref, b1_ref, b2_ref, b3_ref,
               o_ref):
    hi = jax.lax.Precision.HIGHEST
    dg = lambda a, b: jax.lax.dot_general(
        a, b, (((1,), (0,)), ((), ())),
        preferred_element_type=jnp.float32, precision=hi)
    t1 = jnp.maximum(dg(x_ref[...], m1_ref[...]) + b1_ref[0, :][None, :], 0.0)
    t2 = jnp.maximum(dg(t1, m2_ref[...]) + b2_ref[0, :][None, :], 0.0)
    t3 = jnp.maximum(dg(t2, m3_ref[...]) + b3_ref[0, :][None, :], 0.0)
    o_ref[...] = dg(t3, m4_ref[...])[:, 0:1]


def _pad_coo(w, c, r):
    pad = _K - w.shape[0]
    return (jnp.pad(w, (0, pad)).reshape(1, -1),
            jnp.pad(r, (0, pad)).reshape(1, -1),
            jnp.pad(c, (0, pad)).reshape(1, -1))


def kernel(x, sl1_w, sl1_b, fc1_w, fc1_b, sl2_w, sl2_b, fc2_w, fc2_b, fc3_w,
           fc3_b, ro_w, ro_b, rows1, cols1, rows2, cols2):
    b = x.shape[0]
    i32 = jnp.int32

    w1, r1, c1 = _pad_coo(
        jnp.concatenate([sl1_w, fc1_w[0]]),
        jnp.concatenate([cols1, jnp.arange(_L1, dtype=i32)]),
        jnp.concatenate([rows1, jnp.full((_L1,), _L2, i32)]))
    w2, r2, c2 = _pad_coo(
        jnp.concatenate([sl2_w, fc2_w[0], jnp.ones((1,), jnp.float32)]),
        jnp.concatenate([cols2, jnp.arange(_L2, dtype=i32),
                         jnp.full((1,), _L2, i32)]),
        jnp.concatenate([rows2, jnp.full((_L2,), _L3, i32),
                         jnp.full((1,), _L3 + 1, i32)]))
    w3, r3, c3 = _pad_coo(
        jnp.concatenate([fc3_w[0], jnp.ones((2,), jnp.float32)]),
        jnp.concatenate([jnp.arange(_L3, dtype=i32),
                         jnp.asarray([_L3, _L3 + 1], i32)]),
        jnp.concatenate([jnp.zeros((_L3,), i32), jnp.asarray([1, 2], i32)]))
    w4, r4, c4 = _pad_coo(
        jnp.concatenate([ro_w[0, ::-1], ro_b]),
        jnp.arange(4, dtype=i32),
        jnp.zeros((4,), i32))

    zpad = lambda v, off: jnp.pad(v, (off, 128 - off - v.shape[0]))
    b1 = (zpad(sl1_b, 0) + zpad(fc1_b, _L2)).reshape(1, -1)
    b2 = (zpad(sl2_b, 0) + zpad(fc2_b, _L3)).reshape(1, -1)
    b3 = (zpad(fc3_b, 0) + zpad(jnp.ones((1,), jnp.float32), 3)).reshape(1, -1)

    m1, m2, m3, m4 = pl.pallas_call(
        _densify,
        out_shape=[jax.ShapeDtypeStruct((128, 128), jnp.float32)] * 4,
    )(w1, r1, c1, w2, r2, c2, w3, r3, c3, w4, r4, c4)

    small = lambda shp: pl.BlockSpec(shp, lambda i: (0, 0))
    return pl.pallas_call(
        _hnn_block,
        grid=(b // _BM,),
        in_specs=[
            pl.BlockSpec((_BM, _L1), lambda i: (i, 0)),
            small((128, 128)), small((128, 128)),
            small((128, 128)), small((128, 128)),
            small((1, 128)), small((1, 128)), small((1, 128)),
        ],
        out_specs=pl.BlockSpec((_BM, 1), lambda i: (i, 0)),
        out_shape=jax.ShapeDtypeStruct((b, 1), jnp.float32),
    )(x, m1, m2, m3, m4, b1, b2, b3)


# speedup vs baseline: 3.3198x; 3.3198x over previous
import jax
import jax.numpy as jnp
from jax.experimental import pallas as pl

_L1 = 128
_L2 = 64
_L3 = 32
_BM = 2048
_K = 256


def _dense_from_coo(w_ref, rows_ref, cols_ref, in_dim):
    c_iota = jax.lax.broadcasted_iota(jnp.int32, (in_dim, _K), 0)
    cw = jnp.where(cols_ref[0, :][None, :] == c_iota, w_ref[0, :][None, :], 0.0)
    r_iota = jax.lax.broadcasted_iota(jnp.int32, (128, _K), 0)
    r1h = jnp.where(rows_ref[0, :][None, :] == r_iota, 1.0, 0.0)
    return jax.lax.dot_general(
        cw, r1h, (((1,), (1,)), ((), ())),
        preferred_element_type=jnp.float32,
        precision=jax.lax.Precision.HIGHEST,
    ).astype(jnp.bfloat16)


def _densify(w1_ref, r1_ref, c1_ref, w2_ref, r2_ref, c2_ref, w3_ref, r3_ref,
             c3_ref, w4_ref, r4_ref, c4_ref, m1_ref, m2_ref, m3_ref, m4_ref):
    m1_ref[...] = _dense_from_coo(w1_ref, r1_ref, c1_ref, _L1)
    m2_ref[...] = _dense_from_coo(w2_ref, r2_ref, c2_ref, 128)
    m3_ref[...] = _dense_from_coo(w3_ref, r3_ref, c3_ref, 128)
    m4_ref[...] = _dense_from_coo(w4_ref, r4_ref, c4_ref, 128)


def _hnn_block(x_ref, m1_ref, m2_ref, m3_ref, m4_ref, b1_ref, b2_ref, b3_ref,
               o_ref):
    bf = jnp.bfloat16
    dg = lambda a, b: jax.lax.dot_general(
        a, b, (((1,), (0,)), ((), ())), preferred_element_type=jnp.float32)
    xb = x_ref[...].astype(bf)
    t1 = jnp.maximum(dg(xb, m1_ref[...]).astype(bf) + b1_ref[0, :][None, :], 0)
    t2 = jnp.maximum(dg(t1, m2_ref[...]).astype(bf) + b2_ref[0, :][None, :], 0)
    t3 = jnp.maximum(dg(t2, m3_ref[...]).astype(bf) + b3_ref[0, :][None, :], 0)
    o_ref[...] = dg(t3, m4_ref[...])[:, 0:1]


def _pad_coo(w, c, r):
    pad = _K - w.shape[0]
    return (jnp.pad(w, (0, pad)).reshape(1, -1),
            jnp.pad(r, (0, pad)).reshape(1, -1),
            jnp.pad(c, (0, pad)).reshape(1, -1))


def kernel(x, sl1_w, sl1_b, fc1_w, fc1_b, sl2_w, sl2_b, fc2_w, fc2_b, fc3_w,
           fc3_b, ro_w, ro_b, rows1, cols1, rows2, cols2):
    b = x.shape[0]
    i32 = jnp.int32

    w1, r1, c1 = _pad_coo(
        jnp.concatenate([sl1_w, fc1_w[0]]),
        jnp.concatenate([cols1, jnp.arange(_L1, dtype=i32)]),
        jnp.concatenate([rows1, jnp.full((_L1,), _L2, i32)]))
    w2, r2, c2 = _pad_coo(
        jnp.concatenate([sl2_w, fc2_w[0], jnp.ones((1,), jnp.float32)]),
        jnp.concatenate([cols2, jnp.arange(_L2, dtype=i32),
                         jnp.full((1,), _L2, i32)]),
        jnp.concatenate([rows2, jnp.full((_L2,), _L3, i32),
                         jnp.full((1,), _L3 + 1, i32)]))
    w3, r3, c3 = _pad_coo(
        jnp.concatenate([fc3_w[0], jnp.ones((2,), jnp.float32)]),
        jnp.concatenate([jnp.arange(_L3, dtype=i32),
                         jnp.asarray([_L3, _L3 + 1], i32)]),
        jnp.concatenate([jnp.zeros((_L3,), i32), jnp.asarray([1, 2], i32)]))
    w4, r4, c4 = _pad_coo(
        jnp.concatenate([ro_w[0, ::-1], ro_b]),
        jnp.arange(4, dtype=i32),
        jnp.zeros((4,), i32))

    zpad = lambda v, off: jnp.pad(v, (off, 128 - off - v.shape[0]))
    b1 = zpad(sl1_b, 0) + zpad(fc1_b, _L2)
    b2 = zpad(sl2_b, 0) + zpad(fc2_b, _L3)
    b3 = zpad(fc3_b, 0) + zpad(jnp.ones((1,), jnp.float32), 3)
    b1, b2, b3 = (v.astype(jnp.bfloat16).reshape(1, -1) for v in (b1, b2, b3))

    m1, m2, m3, m4 = pl.pallas_call(
        _densify,
        out_shape=[jax.ShapeDtypeStruct((128, 128), jnp.bfloat16)] * 4,
    )(w1, r1, c1, w2, r2, c2, w3, r3, c3, w4, r4, c4)

    small = lambda shp: pl.BlockSpec(shp, lambda i: (0, 0))
    return pl.pallas_call(
        _hnn_block,
        grid=(b // _BM,),
        in_specs=[
            pl.BlockSpec((_BM, _L1), lambda i: (i, 0)),
            small((128, 128)), small((128, 128)),
            small((128, 128)), small((128, 128)),
            small((1, 128)), small((1, 128)), small((1, 128)),
        ],
        out_specs=pl.BlockSpec((_BM, 1), lambda i: (i, 0)),
        out_shape=jax.ShapeDtypeStruct((b, 1), jnp.float32),
    )(x, m1, m2, m3, m4, b1, b2, b3)


# speedup vs baseline: 5.1375x; 1.5475x over previous
import jax
import jax.numpy as jnp
from jax.experimental import pallas as pl
from jax.experimental.pallas import tpu as pltpu

_L1 = 128
_L2 = 64
_L3 = 32
_BM = 16384


def _coo_dense(w, rows, cols, in_dim):
    k = w.shape[0]
    c_iota = jax.lax.broadcasted_iota(jnp.int32, (in_dim, k), 0)
    cw = jnp.where(cols[None, :] == c_iota, w[None, :], 0.0)
    r_iota = jax.lax.broadcasted_iota(jnp.int32, (128, k), 0)
    r1h = jnp.where(rows[None, :] == r_iota, 1.0, 0.0)
    return jax.lax.dot_general(
        cw, r1h, (((1,), (1,)), ((), ())),
        preferred_element_type=jnp.float32,
        precision=jax.lax.Precision.HIGHEST)


def _outer(row_a, row_b):
    return jax.lax.dot_general(
        row_a, row_b, (((0,), (0,)), ((), ())),
        preferred_element_type=jnp.float32,
        precision=jax.lax.Precision.HIGHEST)


def _lane_eq(i):
    return (jax.lax.broadcasted_iota(jnp.int32, (1, 128), 1) == i).astype(
        jnp.float32)


def _cross(c, r):
    ci = jax.lax.broadcasted_iota(jnp.int32, (128, 128), 0)
    ri = jax.lax.broadcasted_iota(jnp.int32, (128, 128), 1)
    return ((ci == c) & (ri == r)).astype(jnp.float32)


def _hnn_body(x_ref, sl1w_ref, sl1b_ref, fc1w_ref, fc1b_ref, sl2w_ref,
              sl2b_ref, fc2w_ref, fc2b_ref, fc3w_ref, fc3b_ref, row_ref,
              rob_ref, rows1_ref, cols1_ref, rows2_ref, cols2_ref, o_ref,
              m1_s, m2_s, m3_s, m4_s, b1_s):
    bf = jnp.bfloat16

    @pl.when(pl.program_id(0) == 0)
    def _densify():
        m1 = (_coo_dense(sl1w_ref[:], rows1_ref[:], cols1_ref[:], _L1)
              + _outer(fc1w_ref[...], _lane_eq(_L2)))
        m1_s[...] = m1.astype(bf)
        b1 = jnp.concatenate([sl1b_ref[:], fc1b_ref[:],
                              jnp.ones((1,), jnp.float32),
                              jnp.zeros((62,), jnp.float32)])
        b1_s[...] = b1.reshape(1, 128).astype(bf)
        fc2p = jnp.concatenate([fc2w_ref[...],
                                jnp.zeros((1, 64), jnp.float32)], axis=1)
        b2row = jnp.concatenate(
            [sl2b_ref[:], fc2b_ref[:], jnp.zeros((1,), jnp.float32),
             jnp.ones((1,), jnp.float32), jnp.zeros((93,), jnp.float32)])
        m2 = (_coo_dense(sl2w_ref[:], rows2_ref[:], cols2_ref[:], 128)
              + _outer(fc2p, _lane_eq(_L3))
              + _cross(_L2, _L3 + 1)
              + _outer(_lane_eq(65), b2row.reshape(1, 128)))
        m2_s[...] = m2.astype(bf)
        fc3p = jnp.concatenate([fc3w_ref[...],
                                jnp.zeros((1, 96), jnp.float32)], axis=1)
        b3row = fc3b_ref[0] * _lane_eq(0) + _lane_eq(3)
        m3 = (_outer(fc3p, _lane_eq(0)) + _cross(_L3, 1) + _cross(_L3 + 1, 2)
              + _outer(_lane_eq(_L3 + 2), b3row))
        m3_s[...] = m3.astype(bf)
        rline = (row_ref[0, 2] * _lane_eq(0) + row_ref[0, 1] * _lane_eq(1)
                 + row_ref[0, 0] * _lane_eq(2) + rob_ref[0] * _lane_eq(3))
        m4_s[...] = _outer(rline, _lane_eq(0)).astype(bf)

    dg = lambda a, b: jax.lax.dot_general(
        a, b, (((1,), (0,)), ((), ())), preferred_element_type=jnp.float32)
    xb = x_ref[...].astype(bf)
    t1 = jnp.maximum(dg(xb, m1_s[...]).astype(bf) + b1_s[0, :][None, :], 0)
    t2 = jnp.maximum(dg(t1, m2_s[...]).astype(bf), 0)
    t3 = jnp.maximum(dg(t2, m3_s[...]).astype(bf), 0)
    o_ref[...] = dg(t3, m4_s[...])[:, 0:1]


def kernel(x, sl1_w, sl1_b, fc1_w, fc1_b, sl2_w, sl2_b, fc2_w, fc2_b, fc3_w,
           fc3_b, ro_w, ro_b, rows1, cols1, rows2, cols2):
    b = x.shape[0]
    full = lambda shp: pl.BlockSpec(shp, (lambda i: (0,) * len(shp)))
    return pl.pallas_call(
        _hnn_body,
        grid=(b // _BM,),
        in_specs=[
            pl.BlockSpec((_BM, _L1), lambda i: (i, 0)),
            full((_L1,)), full((_L2,)), full((1, _L1)), full((1,)),
            full((_L2,)), full((_L3,)), full((1, _L2)), full((1,)),
            full((1, _L3)), full((1,)), full((1, 3)), full((1,)),
            full((_L1,)), full((_L1,)), full((_L2,)), full((_L2,)),
        ],
        out_specs=pl.BlockSpec((_BM, 1), lambda i: (i, 0)),
        out_shape=jax.ShapeDtypeStruct((b, 1), jnp.float32),
        scratch_shapes=[pltpu.VMEM((128, 128), jnp.bfloat16)] * 4
        + [pltpu.VMEM((1, 128), jnp.bfloat16)],
    )(x, sl1_w, sl1_b, fc1_w, fc1_b, sl2_w, sl2_b, fc2_w, fc2_b, fc3_w,
      fc3_b, ro_w, ro_b, rows1, cols1, rows2, cols2)
